# baseline (speedup 1.0000x reference)
import functools

import jax
import jax.numpy as jnp
from jax import lax
from jax.experimental import pallas as pl
from jax.experimental.pallas import tpu as pltpu
from jax.experimental.pallas import tpu_sc as plsc

B = 16384
D = 64
L = 16
NC = 2
NS = 16
NW = NC * NS
BPW = B // NW


def _gmf_body(user_hbm, item_hbm, uemb_hbm, iemb_hbm, w_hbm, bias_hbm,
              out_hbm,
              uidx_v, iidx_v, urows_v, irows_v, w_v, bias_v, out_v,
              sem_u, sem_i):
    wid = lax.axis_index("s") * NC + lax.axis_index("c")
    base = wid * BPW

    pltpu.sync_copy(user_hbm.at[pl.ds(base, BPW)], uidx_v)
    pltpu.sync_copy(item_hbm.at[pl.ds(base, BPW)], iidx_v)
    cu = pltpu.async_copy(uemb_hbm.at[uidx_v], urows_v, sem_u)
    ci = pltpu.async_copy(iemb_hbm.at[iidx_v], irows_v, sem_i)
    pltpu.sync_copy(w_hbm, w_v)
    pltpu.sync_copy(bias_hbm, bias_v)
    cu.wait()
    ci.wait()

    w0 = w_v[pl.ds(0, L)]
    w1 = w_v[pl.ds(L, L)]
    w2 = w_v[pl.ds(2 * L, L)]
    w3 = w_v[pl.ds(3 * L, L)]
    bias = bias_v[...]
    lane = lax.broadcasted_iota(jnp.int32, (L,), 0)
    perms = [lane ^ (L >> (s + 1)) for s in range(4)]

    dnums = lax.GatherDimensionNumbers(
        offset_dims=(), collapsed_slice_dims=(0,), start_index_map=(0,))

    def shuffle(p, perm):
        return lax.gather(p, perm[:, None], dnums, (1,),
                          mode=lax.GatherScatterMode.PROMISE_IN_BOUNDS)

    def lanesum(p):
        for perm in perms:
            p = p + shuffle(p, perm)
        return p

    def group(g, _):
        def one_row(k, acc):
            b = g * L + k
            p = (urows_v[b, pl.ds(0, L)] * irows_v[b, pl.ds(0, L)] * w0
                 + urows_v[b, pl.ds(L, L)] * irows_v[b, pl.ds(L, L)] * w1
                 + urows_v[b, pl.ds(2 * L, L)] * irows_v[b, pl.ds(2 * L, L)] * w2
                 + urows_v[b, pl.ds(3 * L, L)] * irows_v[b, pl.ds(3 * L, L)] * w3)
            tot = lanesum(p)
            return jnp.where(lane == k, tot, acc)

        accv = lax.fori_loop(0, L, one_row, bias)
        out_v[pl.ds(g * L, L)] = accv
        return _

    lax.fori_loop(0, BPW // L, group, 0)
    pltpu.sync_copy(out_v, out_hbm.at[pl.ds(base, BPW)])


@jax.jit
def kernel(user, item, mf_user_embed, mf_item_embed, final_w, final_b):
    w_flat = final_w.reshape(D)
    bias16 = jnp.tile(final_b.reshape(1), L)
    mesh = plsc.VectorSubcoreMesh(core_axis_name="c", subcore_axis_name="s")
    run = functools.partial(
        pl.kernel,
        mesh=mesh,
        compiler_params=pltpu.CompilerParams(use_tc_tiling_on_sc=False),
        out_type=jax.ShapeDtypeStruct((B,), jnp.float32),
        scratch_types=[
            pltpu.VMEM((BPW,), jnp.int32),
            pltpu.VMEM((BPW,), jnp.int32),
            pltpu.VMEM((BPW, D), jnp.float32),
            pltpu.VMEM((BPW, D), jnp.float32),
            pltpu.VMEM((D,), jnp.float32),
            pltpu.VMEM((L,), jnp.float32),
            pltpu.VMEM((BPW,), jnp.float32),
            pltpu.SemaphoreType.DMA,
            pltpu.SemaphoreType.DMA,
        ],
    )(_gmf_body)
    out = run(user.astype(jnp.int32), item.astype(jnp.int32),
              mf_user_embed, mf_item_embed, w_flat, bias16)
    return out.reshape(B, 1)
